# Initial kernel scaffold; baseline (speedup 1.0000x reference)
import jax
import jax.numpy as jnp
from jax.experimental import pallas as pl


def kernel(x, edge_index, edge_attr, edge_src, edge_dst, W1, b1, W2, b2, W3, b3, W4, b4):
    raise NotImplementedError("write your pallas kernel here")


# speedup vs baseline: 11.0363x; 11.0363x over previous
import functools

import jax
import jax.numpy as jnp
from jax import lax
from jax.experimental import pallas as pl
from jax.experimental.pallas import tpu as pltpu
from jax.experimental.pallas import tpu_sc as plsc

N = 10000
E = 320000
D = 128
NC = 2
NS = 16
NW = NC * NS
EPT = E // NW
NPAD = 10240
SPT = NPAD // NS

K1 = 100
NB1 = EPT // K1
K2 = 80
NB2 = EPT // K2

_mesh = plsc.VectorSubcoreMesh(core_axis_name="c", subcore_axis_name="s")


def _sc_degree(dst3d):
    @functools.partial(
        pl.kernel,
        mesh=_mesh,
        compiler_params=pltpu.CompilerParams(use_tc_tiling_on_sc=False),
        out_type=jax.ShapeDtypeStruct((NC, NPAD, 16), jnp.float32),
        scratch_types=[
            pltpu.VMEM((NB1, K1), jnp.int32),
            pltpu.VMEM((K1, 16), jnp.float32),
            pltpu.VMEM((128, 16), jnp.float32),
            pltpu.VMEM_SHARED((NPAD, 16), jnp.float32),
        ],
    )
    def kfn(dst_hbm, out_hbm, idx_v, ones_v, zb_v, acc_s):
        c = lax.axis_index("c")
        s = lax.axis_index("s")
        wid = c * NS + s
        one = jnp.full((16,), 1.0, jnp.float32)
        zero = jnp.zeros((16,), jnp.float32)

        @pl.loop(0, K1)
        def _(i):
            ones_v[i, :] = one

        @pl.loop(0, 128)
        def _(i):
            zb_v[i, :] = zero

        base = s * SPT
        for r in range(SPT // 128):
            pltpu.sync_copy(zb_v, acc_s.at[pl.ds(base + r * 128, 128), :])
        plsc.subcore_barrier()

        pltpu.sync_copy(dst_hbm.at[wid], idx_v)

        @pl.loop(0, NB1)
        def _(j):
            pltpu.sync_copy(ones_v, acc_s.at[idx_v.at[j]], add=True)

        plsc.subcore_barrier()
        pltpu.sync_copy(acc_s.at[pl.ds(base, SPT), :],
                        out_hbm.at[c, pl.ds(base, SPT), :])

    return kfn(dst3d)


def _sc_spmm(g_lo, g_hi, src3d, dst3d):
    HD = 64

    @functools.partial(
        pl.kernel,
        mesh=_mesh,
        compiler_params=pltpu.CompilerParams(use_tc_tiling_on_sc=False),
        out_type=jax.ShapeDtypeStruct((2, NC, NPAD, HD), jnp.float32),
        scratch_types=[
            pltpu.VMEM((NB1, K1), jnp.int32),
            pltpu.VMEM((NB1, K1), jnp.int32),
            pltpu.VMEM((2, K1, HD), jnp.float32),
            pltpu.VMEM((128, HD), jnp.float32),
            pltpu.VMEM_SHARED((NPAD, HD), jnp.float32),
            pltpu.SemaphoreType.DMA,
            pltpu.SemaphoreType.DMA,
        ],
    )
    def kfn(glo_hbm, ghi_hbm, src_hbm, dst_hbm, out_hbm,
            sidx, didx, rows, zb, acc_s, sem0, sem1):
        c = lax.axis_index("c")
        s = lax.axis_index("s")
        wid = c * NS + s
        zero = jnp.zeros((16,), jnp.float32)

        @pl.loop(0, 128)
        def _(i):
            for kk in range(HD // 16):
                zb[i, pl.ds(kk * 16, 16)] = zero

        base = s * SPT
        pltpu.sync_copy(src_hbm.at[wid], sidx)
        pltpu.sync_copy(dst_hbm.at[wid], didx)
        sems = (sem0, sem1)

        for p, g_hbm in enumerate((glo_hbm, ghi_hbm)):
            for r in range(SPT // 128):
                pltpu.sync_copy(zb, acc_s.at[pl.ds(base + r * 128, 128), :])
            plsc.subcore_barrier()

            pltpu.async_copy(g_hbm.at[sidx.at[0]], rows.at[0], sem0)

            @pl.loop(0, NB1, step=2)
            def _(j):
                for b in range(2):
                    jj = j + b
                    nxt = jj + 1

                    @pl.when(nxt < NB1)
                    def _():
                        pltpu.async_copy(g_hbm.at[sidx.at[nxt]],
                                         rows.at[1 - b], sems[1 - b])

                    pltpu.make_async_copy(g_hbm.at[sidx.at[jj]],
                                          rows.at[b], sems[b]).wait()
                    pltpu.sync_copy(rows.at[b], acc_s.at[didx.at[jj]],
                                    add=True)

            plsc.subcore_barrier()
            pltpu.sync_copy(acc_s.at[pl.ds(base, SPT), :],
                            out_hbm.at[p, c, pl.ds(base, SPT), :])

    return kfn(g_lo, g_hi, src3d, dst3d)


def _sc_edge_gather(P, Q, src3d, dst3d):
    H = 64

    @functools.partial(
        pl.kernel,
        mesh=_mesh,
        compiler_params=pltpu.CompilerParams(use_tc_tiling_on_sc=False),
        out_type=jax.ShapeDtypeStruct((E, H), jnp.float32),
        scratch_types=[
            pltpu.VMEM((NB2, K2), jnp.int32),
            pltpu.VMEM((NB2, K2), jnp.int32),
            pltpu.VMEM((2, K2, H), jnp.float32),
            pltpu.VMEM((2, K2, H), jnp.float32),
            pltpu.SemaphoreType.DMA,
            pltpu.SemaphoreType.DMA,
            pltpu.SemaphoreType.DMA,
            pltpu.SemaphoreType.DMA,
        ],
    )
    def kfn(p_hbm, q_hbm, src_hbm, dst_hbm, out_hbm,
            sidx, didx, pb, qb, sp0, sp1, sq0, sq1):
        c = lax.axis_index("c")
        s = lax.axis_index("s")
        wid = c * NS + s

        pltpu.sync_copy(src_hbm.at[wid], sidx)
        pltpu.sync_copy(dst_hbm.at[wid], didx)

        psems = (sp0, sp1)
        qsems = (sq0, sq1)
        pltpu.async_copy(p_hbm.at[sidx.at[0]], pb.at[0], sp0)
        pltpu.async_copy(q_hbm.at[didx.at[0]], qb.at[0], sq0)
        ebase = wid * EPT

        @pl.loop(0, NB2, step=2)
        def _(j):
            for b in range(2):
                jj = j + b
                nxt = jj + 1

                @pl.when(jj < NB2)
                def _():
                    @pl.when(nxt < NB2)
                    def _():
                        pltpu.async_copy(p_hbm.at[sidx.at[nxt]],
                                         pb.at[1 - b], psems[1 - b])
                        pltpu.async_copy(q_hbm.at[didx.at[nxt]],
                                         qb.at[1 - b], qsems[1 - b])

                    pltpu.make_async_copy(p_hbm.at[sidx.at[jj]],
                                          pb.at[b], psems[b]).wait()
                    pltpu.make_async_copy(q_hbm.at[didx.at[jj]],
                                          qb.at[b], qsems[b]).wait()

                    @pl.loop(0, K2)
                    def _(r):
                        for kk in range(H // 16):
                            sl = pl.ds(kk * 16, 16)
                            pb[b, r, sl] = pb[b, r, sl] + qb[b, r, sl]

                    pltpu.sync_copy(pb.at[b],
                                    out_hbm.at[pl.ds(ebase + jj * K2, K2), :])

    return kfn(P, Q, src3d, dst3d)


_BN = 1000


def _tc_scale_in(x, degp, W1):
    def body(x_ref, degp_ref, w_ref, glo_ref, ghi_ref, dinv_ref):
        deg = degp_ref[0] + degp_ref[1] + 1.0
        dinv = lax.rsqrt(deg)
        dinv_ref[...] = dinv
        g = jnp.dot(x_ref[...], w_ref[...],
                    preferred_element_type=jnp.float32) * dinv[:, 0:1]
        glo_ref[...] = g[:, :64]
        ghi_ref[...] = g[:, 64:]

    return pl.pallas_call(
        body,
        grid=(N // _BN,),
        in_specs=[
            pl.BlockSpec((_BN, D), lambda i: (i, 0)),
            pl.BlockSpec((NC, _BN, 16), lambda i: (0, i, 0)),
            pl.BlockSpec((D, D), lambda i: (0, 0)),
        ],
        out_specs=[
            pl.BlockSpec((_BN, 64), lambda i: (i, 0)),
            pl.BlockSpec((_BN, 64), lambda i: (i, 0)),
            pl.BlockSpec((_BN, 16), lambda i: (i, 0)),
        ],
        out_shape=[
            jax.ShapeDtypeStruct((N, 64), jnp.float32),
            jax.ShapeDtypeStruct((N, 64), jnp.float32),
            jax.ShapeDtypeStruct((N, 16), jnp.float32),
        ],
    )(x, degp, W1)


def _tc_mid(acc, glo, ghi, dinv16, W, bprev, relu):
    def body(acc_ref, glo_ref, ghi_ref, dinv_ref, w_ref, b_ref,
             outlo_ref, outhi_ref):
        dinv = dinv_ref[:, 0:1]
        hlo = acc_ref[0, 0] + acc_ref[0, 1] + glo_ref[...]
        hhi = acc_ref[1, 0] + acc_ref[1, 1] + ghi_ref[...]
        h = jnp.concatenate([hlo, hhi], axis=1) * dinv + b_ref[...]
        if relu:
            h = jnp.maximum(h, 0.0)
        out = jnp.dot(h, w_ref[...],
                      preferred_element_type=jnp.float32) * dinv
        outlo_ref[...] = out[:, :64]
        outhi_ref[...] = out[:, 64:]

    return pl.pallas_call(
        body,
        grid=(N // _BN,),
        in_specs=[
            pl.BlockSpec((2, NC, _BN, 64), lambda i: (0, 0, i, 0)),
            pl.BlockSpec((_BN, 64), lambda i: (i, 0)),
            pl.BlockSpec((_BN, 64), lambda i: (i, 0)),
            pl.BlockSpec((_BN, 16), lambda i: (i, 0)),
            pl.BlockSpec((D, D), lambda i: (0, 0)),
            pl.BlockSpec((1, D), lambda i: (0, 0)),
        ],
        out_specs=[
            pl.BlockSpec((_BN, 64), lambda i: (i, 0)),
            pl.BlockSpec((_BN, 64), lambda i: (i, 0)),
        ],
        out_shape=[
            jax.ShapeDtypeStruct((N, 64), jnp.float32),
            jax.ShapeDtypeStruct((N, 64), jnp.float32),
        ],
    )(acc, glo, ghi, dinv16, W, bprev)


def _tc_node_head(acc, glo, ghi, dinv16, W3a, W3b, b2r):
    def body(acc_ref, glo_ref, ghi_ref, dinv_ref, wa_ref, wb_ref, b_ref,
             p_ref, q_ref):
        dinv = dinv_ref[:, 0:1]
        hlo = acc_ref[0, 0] + acc_ref[0, 1] + glo_ref[...]
        hhi = acc_ref[1, 0] + acc_ref[1, 1] + ghi_ref[...]
        h = jnp.concatenate([hlo, hhi], axis=1) * dinv + b_ref[...]
        p_ref[...] = jnp.dot(h, wa_ref[...], preferred_element_type=jnp.float32)
        q_ref[...] = jnp.dot(h, wb_ref[...], preferred_element_type=jnp.float32)

    return pl.pallas_call(
        body,
        grid=(N // _BN,),
        in_specs=[
            pl.BlockSpec((2, NC, _BN, 64), lambda i: (0, 0, i, 0)),
            pl.BlockSpec((_BN, 64), lambda i: (i, 0)),
            pl.BlockSpec((_BN, 64), lambda i: (i, 0)),
            pl.BlockSpec((_BN, 16), lambda i: (i, 0)),
            pl.BlockSpec((D, 64), lambda i: (0, 0)),
            pl.BlockSpec((D, 64), lambda i: (0, 0)),
            pl.BlockSpec((1, D), lambda i: (0, 0)),
        ],
        out_specs=[
            pl.BlockSpec((_BN, 64), lambda i: (i, 0)),
            pl.BlockSpec((_BN, 64), lambda i: (i, 0)),
        ],
        out_shape=[
            jax.ShapeDtypeStruct((N, 64), jnp.float32),
            jax.ShapeDtypeStruct((N, 64), jnp.float32),
        ],
    )(acc, glo, ghi, dinv16, W3a, W3b, b2r)


def _tc_edge_head(G, edge_attr, W3c, b3r, W4, b4r):
    BE = 2000

    def body(g_ref, ea_ref, wc_ref, b3_ref, w4_ref, b4_ref, out_ref):
        z = g_ref[...] + jnp.dot(ea_ref[...], wc_ref[...],
                                 preferred_element_type=jnp.float32) + b3_ref[...]
        z = jnp.maximum(z, 0.0)
        out_ref[...] = jnp.dot(z, w4_ref[...],
                               preferred_element_type=jnp.float32) + b4_ref[...]

    return pl.pallas_call(
        body,
        grid=(E // BE,),
        in_specs=[
            pl.BlockSpec((BE, 64), lambda i: (i, 0)),
            pl.BlockSpec((BE, 16), lambda i: (i, 0)),
            pl.BlockSpec((16, 64), lambda i: (0, 0)),
            pl.BlockSpec((1, 64), lambda i: (0, 0)),
            pl.BlockSpec((64, 2), lambda i: (0, 0)),
            pl.BlockSpec((1, 2), lambda i: (0, 0)),
        ],
        out_specs=pl.BlockSpec((BE, 2), lambda i: (i, 0)),
        out_shape=jax.ShapeDtypeStruct((E, 2), jnp.float32),
    )(G, edge_attr, W3c, b3r, W4, b4r)


def kernel(x, edge_index, edge_attr, edge_src, edge_dst,
           W1, b1, W2, b2, W3, b3, W4, b4):
    src3d = edge_index[0].astype(jnp.int32).reshape(NW, NB1, K1)
    dst3d = edge_index[1].astype(jnp.int32).reshape(NW, NB1, K1)
    es3d = edge_src.astype(jnp.int32).reshape(NW, NB2, K2)
    ed3d = edge_dst.astype(jnp.int32).reshape(NW, NB2, K2)
    W3a = W3[:D]
    W3b = W3[D:2 * D]
    W3c = W3[2 * D:]

    degp = _sc_degree(dst3d)
    g1lo, g1hi, dinv16 = _tc_scale_in(x, degp, W1)
    acc1 = _sc_spmm(g1lo, g1hi, src3d, dst3d)
    g2lo, g2hi = _tc_mid(acc1, g1lo, g1hi, dinv16, W2,
                         b1.reshape(1, D), relu=True)
    acc2 = _sc_spmm(g2lo, g2hi, src3d, dst3d)
    P, Q = _tc_node_head(acc2, g2lo, g2hi, dinv16, W3a, W3b, b2.reshape(1, D))
    G = _sc_edge_gather(P, Q, es3d, ed3d)
    return _tc_edge_head(G, edge_attr, W3c, b3.reshape(1, 64),
                         W4, b4.reshape(1, 2))
